# baseline (speedup 1.0000x reference)
import jax
import jax.numpy as jnp
from jax.experimental import pallas as pl

_NEG = -2.3819763e+38


def _mask_kernel(o_ref):
    i = pl.program_id(0)
    j = pl.program_id(1)
    br = o_ref.shape[2]
    bc = o_ref.shape[3]

    @pl.when(i > j)
    def _():
        o_ref[0, 0, :, :] = jnp.zeros((br, bc), jnp.float32)

    @pl.when(i < j)
    def _():
        o_ref[0, 0, :, :] = jnp.full((br, bc), _NEG, jnp.float32)

    @pl.when(i == j)
    def _():
        rows = jax.lax.broadcasted_iota(jnp.int32, (br, bc), 0)
        cols = jax.lax.broadcasted_iota(jnp.int32, (br, bc), 1)
        o_ref[0, 0, :, :] = jnp.where(cols > rows, _NEG, 0.0).astype(jnp.float32)


def kernel(forward, batch_size, data, device, temperature, top_p, top_k, kv_caches, output_len, is_str_prompt):
    S = data.shape[-2]
    BR = 512
    BC = 512
    grid = (S // BR, S // BC)
    return pl.pallas_call(
        _mask_kernel,
        grid=grid,
        out_specs=pl.BlockSpec((1, 1, BR, BC), lambda i, j: (0, 0, i, j)),
        out_shape=jax.ShapeDtypeStruct((1, 1, S, S), jnp.float32),
    )()


# speedup vs baseline: 1.5506x; 1.5506x over previous
import jax
import jax.numpy as jnp
from jax.experimental import pallas as pl

_NEG = -2.3819763e+38


def _mask_kernel(o_ref):
    i = pl.program_id(0)
    br = o_ref.shape[2]
    s = o_ref.shape[3]
    rows = jax.lax.broadcasted_iota(jnp.int32, (br, s), 0) + i * br
    cols = jax.lax.broadcasted_iota(jnp.int32, (br, s), 1)
    o_ref[0, 0, :, :] = jnp.where(cols > rows, _NEG, 0.0).astype(jnp.float32)


def kernel(forward, batch_size, data, device, temperature, top_p, top_k, kv_caches, output_len, is_str_prompt):
    S = data.shape[-2]
    BR = 512
    grid = (S // BR,)
    return pl.pallas_call(
        _mask_kernel,
        grid=grid,
        out_specs=pl.BlockSpec((1, 1, BR, S), lambda i: (0, 0, i, 0)),
        out_shape=jax.ShapeDtypeStruct((1, 1, S, S), jnp.float32),
    )()
